# Initial kernel scaffold; baseline (speedup 1.0000x reference)
import jax
import jax.numpy as jnp
from jax.experimental import pallas as pl


def kernel(nodes, adjs, W1_l, W1_r, b1, W2_l, W2_r, b2, W3_l, W3_r, b3):
    raise NotImplementedError("write your pallas kernel here")


# speedup vs baseline: 1.1344x; 1.1344x over previous
import jax
import jax.numpy as jnp
from jax.experimental import pallas as pl
from jax.experimental.pallas import tpu as pltpu

_N = 4096
_C = 128
_BLK = 512
_NBLK = _N // _BLK
_HI = jax.lax.Precision.HIGHEST


def _dot_t(a, b, precision=None):
    return jax.lax.dot_general(
        a, b, (((0,), (0,)), ((), ())),
        preferred_element_type=jnp.float32, precision=precision)


def _pass1_body(a_ref, naug_hi_ref, naug_lo_ref, atx_ref):
    i = pl.program_id(0)
    ab = a_ref[...].astype(jnp.bfloat16)
    part = _dot_t(ab, naug_hi_ref[...]) + _dot_t(ab, naug_lo_ref[...])
    @pl.when(i == 0)
    def _():
        atx_ref[...] = part
    @pl.when(i > 0)
    def _():
        atx_ref[...] += part


def _mid_body(atx_ref, nodes_ref, w1l_ref, w1r_ref, b1_ref,
              w2l_ref, w2r_ref, b2_ref,
              shi_ref, slo_ref, s_ref, xout_ref, ent_ref, scal_ref):
    atx = atx_ref[...]
    nodes = nodes_ref[...]
    colsum = atx[:, 2:3]
    deg = jnp.maximum(colsum, 1.0)
    agg = atx[:, 0:2] / deg

    x1 = (jnp.dot(agg, w1l_ref[...].T, precision=_HI)
          + jnp.dot(nodes, w1r_ref[...].T, precision=_HI) + b1_ref[...])
    logits = (jnp.dot(agg, w2l_ref[...].T, precision=_HI)
              + jnp.dot(nodes, w2r_ref[...].T, precision=_HI) + b2_ref[...])

    m = jnp.max(logits, axis=-1, keepdims=True)
    e = jnp.exp(logits - m)
    s = e / jnp.sum(e, axis=-1, keepdims=True)

    shi = s.astype(jnp.bfloat16)
    shi_ref[...] = shi
    slo_ref[...] = (s - shi.astype(jnp.float32)).astype(jnp.bfloat16)
    s_ref[...] = s

    ent = -s * jnp.log(s + 1e-15)
    ent_ref[...] = jnp.reshape(jnp.sum(ent) / _N, (1, 1))

    xout_ref[...] = _dot_t(s, x1, precision=_HI)

    g = _dot_t(s, s, precision=_HI)
    gnorm2 = jnp.sum(g * g)
    suma2 = jnp.sum(colsum)
    scal_ref[...] = jnp.concatenate(
        [jnp.reshape(suma2, (1, 1)), jnp.reshape(gnorm2, (1, 1))], axis=1)


def _pass2_body(a_ref, shi_ref, slo_ref, sblk_ref,
                xaug_ref, w3l_ref, w3r_ref, b3_ref, scal_ref,
                x3_ref, arg_ref, ll_ref, adj_acc):
    i = pl.program_id(0)
    ab = a_ref[...].astype(jnp.bfloat16)
    y = (jnp.dot(ab, shi_ref[...], preferred_element_type=jnp.float32)
         + jnp.dot(ab, slo_ref[...], preferred_element_type=jnp.float32))
    part = _dot_t(sblk_ref[...], y, precision=_HI)
    @pl.when(i == 0)
    def _():
        adj_acc[...] = part
    @pl.when(i > 0)
    def _():
        adj_acc[...] += part

    @pl.when(i == _NBLK - 1)
    def _():
        adj_p = adj_acc[...]
        rows = jax.lax.broadcasted_iota(jnp.int32, (_C, _C), 0)
        cols = jax.lax.broadcasted_iota(jnp.int32, (_C, _C), 1)

        tr = jnp.sum(jnp.where(rows == cols, adj_p, 0.0))
        suma2 = scal_ref[0, 0]
        gnorm2 = scal_ref[0, 1]
        resid = jnp.maximum(suma2 - 2.0 * tr + gnorm2, 0.0)
        ll_ref[...] = jnp.reshape(jnp.sqrt(resid) / (_N * _N), (1, 1))

        row_max = jnp.max(adj_p, axis=1, keepdims=True)
        is_max = adj_p == row_max
        hard = is_max.astype(jnp.float32)
        arg_ref[...] = jnp.min(jnp.where(is_max, cols, _C), axis=1,
                               keepdims=True)

        xaug = xaug_ref[...]
        agg_aug = _dot_t(hard, xaug, precision=_HI)
        deg3 = jnp.maximum(agg_aug[:, 2:3], 1.0)
        agg3 = agg_aug[:, 0:2] / deg3
        x3_ref[...] = (jnp.dot(agg3, w3l_ref[...].T, precision=_HI)
                       + jnp.dot(xaug[:, 0:2], w3r_ref[...].T, precision=_HI)
                       + b3_ref[...])


def kernel(nodes, adjs, W1_l, W1_r, b1, W2_l, W2_r, b2, W3_l, W3_r, b3):
    naug = jnp.concatenate(
        [nodes, jnp.ones((_N, 1), jnp.float32)], axis=1)
    naug_hi = naug.astype(jnp.bfloat16)
    naug_lo = (naug - naug_hi.astype(jnp.float32)).astype(jnp.bfloat16)

    atx = pl.pallas_call(
        _pass1_body,
        grid=(_NBLK,),
        in_specs=[
            pl.BlockSpec((_BLK, _N), lambda i: (i, 0)),
            pl.BlockSpec((_BLK, 3), lambda i: (i, 0)),
            pl.BlockSpec((_BLK, 3), lambda i: (i, 0)),
        ],
        out_specs=pl.BlockSpec((_N, 3), lambda i: (0, 0)),
        out_shape=jax.ShapeDtypeStruct((_N, 3), jnp.float32),
        compiler_params=pltpu.CompilerParams(
            dimension_semantics=("arbitrary",)),
    )(adjs, naug_hi, naug_lo)

    shi, slo, s, xout, ent, scal = pl.pallas_call(
        _mid_body,
        out_shape=[
            jax.ShapeDtypeStruct((_N, _C), jnp.bfloat16),
            jax.ShapeDtypeStruct((_N, _C), jnp.bfloat16),
            jax.ShapeDtypeStruct((_N, _C), jnp.float32),
            jax.ShapeDtypeStruct((_C, 2), jnp.float32),
            jax.ShapeDtypeStruct((1, 1), jnp.float32),
            jax.ShapeDtypeStruct((1, 2), jnp.float32),
        ],
    )(atx, nodes, W1_l, W1_r, b1.reshape(1, 2), W2_l, W2_r, b2.reshape(1, _C))

    xaug = jnp.concatenate(
        [xout, jnp.ones((_C, 1), jnp.float32)], axis=1)

    x3, arg, ll = pl.pallas_call(
        _pass2_body,
        grid=(_NBLK,),
        in_specs=[
            pl.BlockSpec((_BLK, _N), lambda i: (i, 0)),
            pl.BlockSpec((_N, _C), lambda i: (0, 0)),
            pl.BlockSpec((_N, _C), lambda i: (0, 0)),
            pl.BlockSpec((_BLK, _C), lambda i: (i, 0)),
            pl.BlockSpec((_C, 3), lambda i: (0, 0)),
            pl.BlockSpec((1, 2), lambda i: (0, 0)),
            pl.BlockSpec((1, 2), lambda i: (0, 0)),
            pl.BlockSpec((1, 1), lambda i: (0, 0)),
            pl.BlockSpec((1, 2), lambda i: (0, 0)),
        ],
        out_specs=[
            pl.BlockSpec((_C, 1), lambda i: (0, 0)),
            pl.BlockSpec((_C, 1), lambda i: (0, 0)),
            pl.BlockSpec((1, 1), lambda i: (0, 0)),
        ],
        out_shape=[
            jax.ShapeDtypeStruct((_C, 1), jnp.float32),
            jax.ShapeDtypeStruct((_C, 1), jnp.int32),
            jax.ShapeDtypeStruct((1, 1), jnp.float32),
        ],
        scratch_shapes=[pltpu.VMEM((_C, _C), jnp.float32)],
        compiler_params=pltpu.CompilerParams(
            dimension_semantics=("arbitrary",)),
    )(adjs, shi, slo, s, xaug, W3_l, W3_r, b3.reshape(1, 1), scal)

    edge_index = jnp.stack(
        [jnp.arange(_C, dtype=jnp.int32), arg.reshape(_C)])
    return (x3.reshape(_C), edge_index,
            ll.reshape(()), ent.reshape(()), xout)


# speedup vs baseline: 1.3490x; 1.1892x over previous
import jax
import jax.numpy as jnp
from jax.experimental import pallas as pl
from jax.experimental.pallas import tpu as pltpu

_N = 4096
_C = 128
_BLK = 512
_NBLK = _N // _BLK


def _dot_t(a, b):
    return jax.lax.dot_general(
        a, b, (((0,), (0,)), ((), ())),
        preferred_element_type=jnp.float32)


def _split(x):
    hi = x.astype(jnp.bfloat16)
    lo = (x - hi.astype(jnp.float32)).astype(jnp.bfloat16)
    return hi, lo


def _lin2(a, b, Wl, Wr, bias):
    return (a[:, 0:1] * Wl[:, 0][None, :] + a[:, 1:2] * Wl[:, 1][None, :]
            + b[:, 0:1] * Wr[:, 0][None, :] + b[:, 1:2] * Wr[:, 1][None, :]
            + bias)


def _body(a_ref, nodes_ref, nhi_ref, nlo_ref,
          w1l_ref, w1r_ref, b1_ref, w2l_ref, w2r_ref, b2_ref,
          w3l_ref, w3r_ref, b3_ref,
          x3_ref, arg_ref, ll_ref, ent_ref, xout_ref,
          abf_ref, atx_ref, shi_ref, slo_ref, adj_ref, scal_ref, xaug_ref):
    k = pl.program_id(0)

    @pl.when(k < _NBLK)
    def _pass1():
        af = a_ref[...]
        abf_ref[pl.ds(k * _BLK, _BLK), :] = af.astype(jnp.int8)
        ab = af.astype(jnp.bfloat16)
        nh = nhi_ref[pl.ds(k * _BLK, _BLK), :]
        nl = nlo_ref[pl.ds(k * _BLK, _BLK), :]
        part = _dot_t(ab, nh) + _dot_t(ab, nl)
        @pl.when(k == 0)
        def _():
            atx_ref[...] = part
        @pl.when(k > 0)
        def _():
            atx_ref[...] += part

    @pl.when(k == _NBLK - 1)
    def _mid():
        atx = atx_ref[...]
        nodes = nodes_ref[...]
        colsum = atx[:, 2:3]
        deg = jnp.maximum(colsum, 1.0)
        agg = atx[:, 0:2] / deg

        x1 = _lin2(agg, nodes, w1l_ref[...], w1r_ref[...], b1_ref[...])
        logits = _lin2(agg, nodes, w2l_ref[...], w2r_ref[...], b2_ref[...])

        m = jnp.max(logits, axis=-1, keepdims=True)
        e = jnp.exp(logits - m)
        z = jnp.sum(e, axis=-1, keepdims=True)
        s = e / z
        shi, slo = _split(s)
        shi_ref[...] = shi
        slo_ref[...] = slo

        ent_rows = jnp.log(z) - jnp.sum(e * (logits - m), axis=-1,
                                        keepdims=True) / z
        ent_ref[...] = jnp.reshape(jnp.sum(ent_rows) / _N, (1, 1))

        x1h, x1l = _split(x1)
        xout = _dot_t(shi, x1h) + _dot_t(shi, x1l) + _dot_t(slo, x1h)
        xout_ref[...] = xout
        xaug_ref[...] = jnp.concatenate(
            [xout, jnp.ones((_C, 1), jnp.float32)], axis=1)

        g = _dot_t(shi, shi) + _dot_t(shi, slo) + _dot_t(slo, shi)
        gnorm2 = jnp.sum(g * g)
        suma2 = jnp.sum(colsum)
        scal_ref[...] = jnp.concatenate(
            [jnp.reshape(suma2, (1, 1)), jnp.reshape(gnorm2, (1, 1))], axis=1)

    @pl.when(k >= _NBLK)
    def _pass2():
        j = k - _NBLK
        ab = abf_ref[pl.ds(j * _BLK, _BLK), :].astype(jnp.bfloat16)
        y = (jnp.dot(ab, shi_ref[...], preferred_element_type=jnp.float32)
             + jnp.dot(ab, slo_ref[...], preferred_element_type=jnp.float32))
        yh, yl = _split(y)
        sh = shi_ref[pl.ds(j * _BLK, _BLK), :]
        sl = slo_ref[pl.ds(j * _BLK, _BLK), :]
        part = _dot_t(sh, yh) + _dot_t(sh, yl) + _dot_t(sl, yh)
        @pl.when(j == 0)
        def _():
            adj_ref[...] = part
        @pl.when(j > 0)
        def _():
            adj_ref[...] += part

    @pl.when(k == 2 * _NBLK - 1)
    def _final():
        adj_p = adj_ref[...]
        rows = jax.lax.broadcasted_iota(jnp.int32, (_C, _C), 0)
        cols = jax.lax.broadcasted_iota(jnp.int32, (_C, _C), 1)

        tr = jnp.sum(jnp.where(rows == cols, adj_p, 0.0))
        suma2 = scal_ref[0, 0]
        gnorm2 = scal_ref[0, 1]
        resid = jnp.maximum(suma2 - 2.0 * tr + gnorm2, 0.0)
        ll_ref[...] = jnp.reshape(jnp.sqrt(resid) / (_N * _N), (1, 1))

        row_max = jnp.max(adj_p, axis=1, keepdims=True)
        is_max = adj_p == row_max
        hard = is_max.astype(jnp.float32)
        arg_ref[...] = jnp.min(jnp.where(is_max, cols, _C), axis=1,
                               keepdims=True)

        xaug = xaug_ref[...]
        agg_aug = _dot_t(hard, xaug)
        deg3 = jnp.maximum(agg_aug[:, 2:3], 1.0)
        agg3 = agg_aug[:, 0:2] / deg3
        x3_ref[...] = _lin2(agg3, xaug[:, 0:2], w3l_ref[...], w3r_ref[...],
                            b3_ref[...])


def kernel(nodes, adjs, W1_l, W1_r, b1, W2_l, W2_r, b2, W3_l, W3_r, b3):
    naug = jnp.concatenate(
        [nodes, jnp.ones((_N, 1), jnp.float32)], axis=1)
    naug_hi = naug.astype(jnp.bfloat16)
    naug_lo = (naug - naug_hi.astype(jnp.float32)).astype(jnp.bfloat16)

    x3, arg, ll, ent, xout = pl.pallas_call(
        _body,
        grid=(2 * _NBLK,),
        in_specs=[
            pl.BlockSpec((_BLK, _N), lambda k: (jnp.minimum(k, _NBLK - 1), 0)),
            pl.BlockSpec((_N, 2), lambda k: (0, 0)),
            pl.BlockSpec((_N, 3), lambda k: (0, 0)),
            pl.BlockSpec((_N, 3), lambda k: (0, 0)),
            pl.BlockSpec((2, 2), lambda k: (0, 0)),
            pl.BlockSpec((2, 2), lambda k: (0, 0)),
            pl.BlockSpec((1, 2), lambda k: (0, 0)),
            pl.BlockSpec((_C, 2), lambda k: (0, 0)),
            pl.BlockSpec((_C, 2), lambda k: (0, 0)),
            pl.BlockSpec((1, _C), lambda k: (0, 0)),
            pl.BlockSpec((1, 2), lambda k: (0, 0)),
            pl.BlockSpec((1, 2), lambda k: (0, 0)),
            pl.BlockSpec((1, 1), lambda k: (0, 0)),
        ],
        out_specs=[
            pl.BlockSpec((_C, 1), lambda k: (0, 0)),
            pl.BlockSpec((_C, 1), lambda k: (0, 0)),
            pl.BlockSpec((1, 1), lambda k: (0, 0)),
            pl.BlockSpec((1, 1), lambda k: (0, 0)),
            pl.BlockSpec((_C, 2), lambda k: (0, 0)),
        ],
        out_shape=[
            jax.ShapeDtypeStruct((_C, 1), jnp.float32),
            jax.ShapeDtypeStruct((_C, 1), jnp.int32),
            jax.ShapeDtypeStruct((1, 1), jnp.float32),
            jax.ShapeDtypeStruct((1, 1), jnp.float32),
            jax.ShapeDtypeStruct((_C, 2), jnp.float32),
        ],
        scratch_shapes=[
            pltpu.VMEM((_N, _N), jnp.int8),
            pltpu.VMEM((_N, 3), jnp.float32),
            pltpu.VMEM((_N, _C), jnp.bfloat16),
            pltpu.VMEM((_N, _C), jnp.bfloat16),
            pltpu.VMEM((_C, _C), jnp.float32),
            pltpu.VMEM((1, 2), jnp.float32),
            pltpu.VMEM((_C, 3), jnp.float32),
        ],
        compiler_params=pltpu.CompilerParams(
            dimension_semantics=("arbitrary",)),
    )(adjs, nodes, naug_hi, naug_lo,
      W1_l, W1_r, b1.reshape(1, 2), W2_l, W2_r, b2.reshape(1, _C),
      W3_l, W3_r, b3.reshape(1, 1))

    x3_out = x3[:, 0]
    edge_index = jnp.stack(
        [jnp.arange(_C, dtype=jnp.int32), arg.reshape(_C)])
    return (x3_out, edge_index, ll.reshape(()), ent.reshape(()), xout)
